# Initial kernel scaffold; baseline (speedup 1.0000x reference)
import jax
import jax.numpy as jnp
from jax.experimental import pallas as pl


def kernel(x, edge_index, edge_attr, w_ih_f, w_hh_f, b_ih_f, b_hh_f, w_ih_b, w_hh_b, b_ih_b, b_hh_b, fc_w, fc_b, w1_init, w1_root, b1, w2_init, w2_root, b2, data):
    raise NotImplementedError("write your pallas kernel here")


# speedup vs baseline: 2.2808x; 2.2808x over previous
import jax
import jax.numpy as jnp
from jax.experimental import pallas as pl

N = 100000
H = 20
L = 12


def _gru_dir(xs, w_ih, w_hh, b_ih, b_hh):
    def step(h, xt):
        gi = xt @ w_ih.T + b_ih
        gh = h @ w_hh.T + b_hh
        ir, iz, inn = jnp.split(gi, 3, axis=-1)
        hr, hz, hn = jnp.split(gh, 3, axis=-1)
        r = jax.nn.sigmoid(ir + hr)
        z = jax.nn.sigmoid(iz + hz)
        n = jnp.tanh(inn + r * hn)
        return (1.0 - z) * n + z * h, None
    h0 = jnp.zeros((xs.shape[1], H), jnp.float32)
    hT, _ = jax.lax.scan(step, h0, xs)
    return hT


def _combine_body(agg_ref, hr_ref, b_ref, o_ref):
    o_ref[...] = jnp.maximum(agg_ref[...] + hr_ref[...] + b_ref[...], 0.0)


def _combine(agg, hr, b):
    B = 4000
    return pl.pallas_call(
        _combine_body,
        grid=(N // B,),
        in_specs=[
            pl.BlockSpec((B, H), lambda i: (i, 0)),
            pl.BlockSpec((B, H), lambda i: (i, 0)),
            pl.BlockSpec((1, H), lambda i: (0, 0)),
        ],
        out_specs=pl.BlockSpec((B, H), lambda i: (i, 0)),
        out_shape=jax.ShapeDtypeStruct((N, H), jnp.float32),
    )(agg, hr, b.reshape(1, H))


def kernel(x, edge_index, edge_attr, w_ih_f, w_hh_f, b_ih_f, b_hh_f,
           w_ih_b, w_hh_b, b_ih_b, b_hh_b, fc_w, fc_b,
           w1_init, w1_root, b1, w2_init, w2_root, b2, data):
    edge_index = edge_index.astype(jnp.int32)
    xs = jnp.transpose(x, (1, 0, 2))
    hf = _gru_dir(xs, w_ih_f, w_hh_f, b_ih_f, b_hh_f)
    hb = _gru_dir(xs[::-1], w_ih_b, w_hh_b, b_ih_b, b_hh_b)
    h_n = jnp.stack([hf, hb], axis=0).reshape(N, 2 * H)
    h = jax.nn.relu(h_n @ fc_w.T + fc_b)

    row, col = edge_index[0], edge_index[1]
    deg = jnp.zeros((N,), jnp.float32).at[col].add(edge_attr)
    dis = jnp.where(deg > 0, jax.lax.rsqrt(jnp.where(deg > 0, deg, 1.0)), 0.0)

    def conv(h, w_init, w_root, b):
        mp = dis[:, None] * (h @ w_init)
        agg_raw = jnp.zeros((N, H), jnp.float32).at[col].add(
            edge_attr[:, None] * mp[row])
        return _combine(dis[:, None] * agg_raw, h @ w_root, b)

    h1 = conv(h, w1_init, w1_root, b1)
    out = conv(h1, w2_init, w2_root, b2)
    return out


# speedup vs baseline: 10.3734x; 4.5482x over previous
import numpy as np
import jax
import jax.numpy as jnp
from jax import lax
from jax.experimental import pallas as pl
from jax.experimental.pallas import tpu as pltpu
from jax.experimental.pallas import tpu_sc as plsc

N = 100000
E = 3200000
L = 12
H = 20
D = 16

RPT = N // 16
EPT = 100352
EPAD = EPT * 32
TPS = EPAD // 16
CHUNK = 128
SUB = 4
WIN = SUB * CHUNK

def _sc_mesh():
    return plsc.VectorSubcoreMesh(core_axis_name="c", subcore_axis_name="s")


_sc_params = pltpu.CompilerParams(use_tc_tiling_on_sc=False,
                                  needs_layout_passes=False)


def _deg_body(col2d, ew2d, z16, dout, cbuf, wbuf, sbuf, deg_sh):
    c = lax.axis_index("c")
    s = lax.axis_index("s")
    wid = s * 2 + c
    r0 = s * RPT
    pltpu.sync_copy(z16.at[pl.ds(r0, RPT)], deg_sh.at[pl.ds(r0, RPT)])
    pltpu.sync_copy(z16.at[pl.ds(0, CHUNK)], sbuf)
    plsc.subcore_barrier()
    qbase = wid * (EPT // CHUNK)
    lane = lax.iota(jnp.int32, 16)
    zero16 = lane * 0

    def step(st, carry):
        q = qbase + st * SUB
        pltpu.sync_copy(col2d.at[pl.ds(q, SUB)], cbuf)
        pltpu.sync_copy(ew2d.at[pl.ds(q, SUB)], wbuf)
        for j in range(SUB):
            for k in range(CHUNK // 16):
                ew16 = wbuf[j, pl.ds(k * 16, 16)]
                plsc.store_scatter(sbuf, [lane + k * 16, zero16], ew16)
            pltpu.sync_copy(sbuf, deg_sh.at[cbuf.at[j]], add=True)
        return carry

    lax.fori_loop(0, EPT // WIN, step, 0)
    plsc.subcore_barrier()
    pltpu.sync_copy(deg_sh.at[pl.ds(r0, RPT)], dout.at[c, pl.ds(r0, RPT)])


def _deg_call(col2d, ew2d, z16):
    return pl.kernel(
        _deg_body,
        out_type=jax.ShapeDtypeStruct((2, N, D), jnp.float32),
        mesh=_sc_mesh(),
        compiler_params=_sc_params,
        scratch_types=[
            pltpu.VMEM((SUB, CHUNK), jnp.int32),
            pltpu.VMEM((SUB, CHUNK), jnp.float32),
            pltpu.VMEM((CHUNK, D), jnp.float32),
            pltpu.VMEM_SHARED((N, D), jnp.float32),
        ],
    )(col2d, ew2d, z16)


def _agg_body(row2, col2d, ew2d, mptab, z16, out, rbuf, cbuf, wbuf, gbuf,
              agg_sh, sem):
    c = lax.axis_index("c")
    s = lax.axis_index("s")
    r0 = s * RPT
    pltpu.sync_copy(z16.at[pl.ds(r0, RPT)], agg_sh.at[pl.ds(r0, RPT)])
    plsc.subcore_barrier()
    base = s * TPS
    zero16 = lax.iota(jnp.int32, 16) * 0

    def step(st, carry):
        off = base + st * WIN
        q = off // CHUNK
        pltpu.sync_copy(row2.at[c, pl.ds(q, SUB)], rbuf)
        pltpu.sync_copy(col2d.at[pl.ds(q, SUB)], cbuf)
        pltpu.sync_copy(ew2d.at[pl.ds(q, SUB)], wbuf)
        for j in range(SUB):
            pltpu.async_copy(mptab.at[rbuf.at[j]], gbuf, sem).wait()
            for e in range(CHUNK):
                ew16 = plsc.load_gather(wbuf, [zero16 + j, zero16 + e])
                gbuf[e, :] = gbuf[e, :] * ew16
            pltpu.sync_copy(gbuf, agg_sh.at[cbuf.at[j]], add=True)
        return carry

    lax.fori_loop(0, TPS // WIN, step, 0)
    plsc.subcore_barrier()
    pltpu.sync_copy(agg_sh.at[pl.ds(r0, RPT)], out.at[c, pl.ds(r0, RPT)])


def _agg_call(row2, col2d, ew2d, mptab, z16):
    return pl.kernel(
        _agg_body,
        out_type=jax.ShapeDtypeStruct((2, N, D), jnp.float32),
        mesh=_sc_mesh(),
        compiler_params=_sc_params,
        scratch_types=[
            pltpu.VMEM((SUB, CHUNK), jnp.int32),
            pltpu.VMEM((SUB, CHUNK), jnp.int32),
            pltpu.VMEM((SUB, CHUNK), jnp.float32),
            pltpu.VMEM((CHUNK, D), jnp.float32),
            pltpu.VMEM_SHARED((N, D), jnp.float32),
            pltpu.SemaphoreType.DMA,
        ],
    )(row2, col2d, ew2d, mptab, z16)


_BE = 2000
_GE = N // _BE


def _dis_of(degp_blk):
    deg = degp_blk[0, :, 0:1] + degp_blk[1, :, 0:1]
    safe = jnp.where(deg > 0, deg, 1.0)
    return jnp.where(deg > 0, lax.rsqrt(safe), 0.0)


def _enc_body(xa, xb, wih, bih, whh, bhh, fcw, fcb, w1, w1r, b1, degp,
              mp_ref, hr_ref):
    x = jnp.concatenate([xa[...], xb[...]], axis=0)
    gi_all = jnp.dot(x, wih[0], preferred_element_type=jnp.float32) + bih[0]
    h = jnp.zeros((2 * _BE, H), jnp.float32)
    w = whh[0]
    bh = bhh[0]
    for t in range(L):
        gi = gi_all[:, 3 * H * t:3 * H * (t + 1)]
        gh = jnp.dot(h, w, preferred_element_type=jnp.float32) + bh
        r = jax.nn.sigmoid(gi[:, :H] + gh[:, :H])
        z = jax.nn.sigmoid(gi[:, H:2 * H] + gh[:, H:2 * H])
        n = jnp.tanh(gi[:, 2 * H:] + r * gh[:, 2 * H:])
        h = (1.0 - z) * n + z * h
    ha = h[:_BE]
    hb = h[_BE:]
    hfc = jax.nn.relu(jnp.dot(ha, fcw[0], preferred_element_type=jnp.float32)
                      + jnp.dot(hb, fcw[1], preferred_element_type=jnp.float32)
                      + fcb[...])
    dis = _dis_of(degp[...])
    mp = dis * jnp.dot(hfc, w1[...], preferred_element_type=jnp.float32)
    mp_ref[0] = mp[:, :D]
    mp_ref[1] = jnp.concatenate(
        [mp[:, D:], jnp.zeros((_BE, 2 * D - H), jnp.float32)], axis=1)
    hr_ref[...] = jnp.dot(hfc, w1r[...], preferred_element_type=jnp.float32) + b1[...]


def _enc_call(xa, xb, wih, bih, whh, bhh, fcw, fcb, w1, w1r, b1, degp):
    g2 = _GE // 2
    return pl.pallas_call(
        _enc_body,
        grid=(_GE,),
        in_specs=[
            pl.BlockSpec((_BE, 2 * L), lambda j: (j, 0)),
            pl.BlockSpec((_BE, 2 * L), lambda j: (j, 0)),
            pl.BlockSpec((1, 2 * L, 3 * H * L), lambda j: (j // g2, 0, 0)),
            pl.BlockSpec((1, 1, 3 * H * L), lambda j: (j // g2, 0, 0)),
            pl.BlockSpec((1, H, 3 * H), lambda j: (j // g2, 0, 0)),
            pl.BlockSpec((1, 1, 3 * H), lambda j: (j // g2, 0, 0)),
            pl.BlockSpec((2, H, H), lambda j: (0, 0, 0)),
            pl.BlockSpec((1, H), lambda j: (0, 0)),
            pl.BlockSpec((H, H), lambda j: (0, 0)),
            pl.BlockSpec((H, H), lambda j: (0, 0)),
            pl.BlockSpec((1, H), lambda j: (0, 0)),
            pl.BlockSpec((2, _BE, D), lambda j: (0, j, 0)),
        ],
        out_specs=[
            pl.BlockSpec((2, _BE, D), lambda j: (0, j, 0)),
            pl.BlockSpec((_BE, H), lambda j: (j, 0)),
        ],
        out_shape=[
            jax.ShapeDtypeStruct((2, N, D), jnp.float32),
            jax.ShapeDtypeStruct((N, H), jnp.float32),
        ],
    )(xa, xb, wih, bih, whh, bhh, fcw, fcb, w1, w1r, b1, degp)


def _agg20(aggp_blk):
    return jnp.concatenate([aggp_blk[0], aggp_blk[1][:, :H - D]], axis=1)


def _c1_body(aggp, degp, hr1, w2, w2r, b2, mp2_ref, hr2_ref):
    dis = _dis_of(degp[...])
    h1 = jax.nn.relu(dis * _agg20(aggp[...]) + hr1[...])
    mp2 = dis * jnp.dot(h1, w2[...], preferred_element_type=jnp.float32)
    mp2_ref[0] = mp2[:, :D]
    mp2_ref[1] = jnp.concatenate(
        [mp2[:, D:], jnp.zeros((_BE, 2 * D - H), jnp.float32)], axis=1)
    hr2_ref[...] = jnp.dot(h1, w2r[...], preferred_element_type=jnp.float32) + b2[...]


def _c1_call(aggp, degp, hr1, w2, w2r, b2):
    return pl.pallas_call(
        _c1_body,
        grid=(_GE,),
        in_specs=[
            pl.BlockSpec((2, _BE, D), lambda j: (0, j, 0)),
            pl.BlockSpec((2, _BE, D), lambda j: (0, j, 0)),
            pl.BlockSpec((_BE, H), lambda j: (j, 0)),
            pl.BlockSpec((H, H), lambda j: (0, 0)),
            pl.BlockSpec((H, H), lambda j: (0, 0)),
            pl.BlockSpec((1, H), lambda j: (0, 0)),
        ],
        out_specs=[
            pl.BlockSpec((2, _BE, D), lambda j: (0, j, 0)),
            pl.BlockSpec((_BE, H), lambda j: (j, 0)),
        ],
        out_shape=[
            jax.ShapeDtypeStruct((2, N, D), jnp.float32),
            jax.ShapeDtypeStruct((N, H), jnp.float32),
        ],
    )(aggp, degp, hr1, w2, w2r, b2)


def _c2_body(aggp, degp, hr2, out_ref):
    dis = _dis_of(degp[...])
    out_ref[...] = jax.nn.relu(dis * _agg20(aggp[...]) + hr2[...])


def _c2_call(aggp, degp, hr2):
    return pl.pallas_call(
        _c2_body,
        grid=(_GE,),
        in_specs=[
            pl.BlockSpec((2, _BE, D), lambda j: (0, j, 0)),
            pl.BlockSpec((2, _BE, D), lambda j: (0, j, 0)),
            pl.BlockSpec((_BE, H), lambda j: (j, 0)),
        ],
        out_specs=pl.BlockSpec((_BE, H), lambda j: (j, 0)),
        out_shape=jax.ShapeDtypeStruct((N, H), jnp.float32),
    )(aggp, degp, hr2)


def _build_gru_weights(w_ih_f, b_ih_f, w_ih_b, b_ih_b, w_hh_f, b_hh_f,
                       w_hh_b, b_hh_b):
    wih = jnp.zeros((2, 2 * L, 3 * H * L), jnp.float32)
    for t in range(L):
        wih = wih.at[0, 2 * t:2 * t + 2, 3 * H * t:3 * H * (t + 1)].set(w_ih_f.T)
        tb = L - 1 - t
        wih = wih.at[1, 2 * tb:2 * tb + 2, 3 * H * t:3 * H * (t + 1)].set(w_ih_b.T)
    bih = jnp.stack([jnp.tile(b_ih_f, L), jnp.tile(b_ih_b, L)])[:, None, :]
    whh = jnp.stack([w_hh_f.T, w_hh_b.T])
    bhh = jnp.stack([b_hh_f, b_hh_b])[:, None, :]
    return wih, bih, whh, bhh


def kernel(x, edge_index, edge_attr, w_ih_f, w_hh_f, b_ih_f, b_hh_f,
           w_ih_b, w_hh_b, b_ih_b, b_hh_b, fc_w, fc_b,
           w1_init, w1_root, b1, w2_init, w2_root, b2, data):
    edge_index = edge_index.astype(jnp.int32)
    rowp = jnp.concatenate([edge_index[0], jnp.zeros((EPAD - E,), jnp.int32)])
    colp = jnp.concatenate([edge_index[1], jnp.zeros((EPAD - E,), jnp.int32)])
    ewp = jnp.concatenate([edge_attr, jnp.zeros((EPAD - E,), jnp.float32)])
    row2 = jnp.stack([rowp, rowp + N]).reshape(
        2, EPAD // CHUNK, CHUNK)
    col2d = colp.reshape(EPAD // CHUNK, CHUNK)
    ew2d = ewp.reshape(EPAD // CHUNK, CHUNK)

    x2d = x.reshape(N, 2 * L)
    xe = x2d[0::2]
    xo = x2d[1::2]
    xa = jnp.concatenate([xe, xe], axis=0)
    xb = jnp.concatenate([xo, xo], axis=0)

    wih, bih, whh, bhh = _build_gru_weights(
        w_ih_f, b_ih_f, w_ih_b, b_ih_b, w_hh_f, b_hh_f, w_hh_b, b_hh_b)
    fcw = jnp.stack([fc_w.T[:H], fc_w.T[H:]])
    fcb = fc_b[None, :]

    z16 = jnp.zeros((N, D), jnp.float32)

    degp = _deg_call(col2d, ew2d, z16)
    mp1, hr1 = _enc_call(xa, xb, wih, bih, whh, bhh, fcw, fcb,
                         w1_init, w1_root, b1[None, :], degp)
    aggp1 = _agg_call(row2, col2d, ew2d, mp1.reshape(2 * N, D), z16)
    mp2, hr2 = _c1_call(aggp1, degp, hr1, w2_init, w2_root, b2[None, :])
    aggp2 = _agg_call(row2, col2d, ew2d, mp2.reshape(2 * N, D), z16)
    return _c2_call(aggp2, degp, hr2)


# speedup vs baseline: 13.2865x; 1.2808x over previous
import numpy as np
import jax
import jax.numpy as jnp
from jax import lax
from jax.experimental import pallas as pl
from jax.experimental.pallas import tpu as pltpu
from jax.experimental.pallas import tpu_sc as plsc

N = 100000
E = 3200000
L = 12
H = 20
D = 16

RPT = N // 16
EPT = 100352
EPAD = EPT * 32
TPS = EPAD // 16
CHUNK = 128
SUB = 4
WIN = SUB * CHUNK

def _sc_mesh():
    return plsc.VectorSubcoreMesh(core_axis_name="c", subcore_axis_name="s")


_sc_params = pltpu.CompilerParams(use_tc_tiling_on_sc=False,
                                  needs_layout_passes=False)


def _deg_body(col2d, ew2d, z16, dout, cbuf, wbuf, sbuf, deg_sh):
    c = lax.axis_index("c")
    s = lax.axis_index("s")
    wid = s * 2 + c
    r0 = s * RPT
    pltpu.sync_copy(z16.at[pl.ds(r0, RPT)], deg_sh.at[pl.ds(r0, RPT)])
    pltpu.sync_copy(z16.at[pl.ds(0, CHUNK)], sbuf)
    plsc.subcore_barrier()
    qbase = wid * (EPT // CHUNK)
    lane = lax.iota(jnp.int32, 16)
    zero16 = lane * 0

    def step(st, carry):
        q = qbase + st * SUB
        pltpu.sync_copy(col2d.at[pl.ds(q, SUB)], cbuf)
        pltpu.sync_copy(ew2d.at[pl.ds(q, SUB)], wbuf)
        for j in range(SUB):
            for k in range(CHUNK // 16):
                ew16 = wbuf[j, pl.ds(k * 16, 16)]
                plsc.store_scatter(sbuf, [lane + k * 16, zero16], ew16)
            pltpu.sync_copy(sbuf, deg_sh.at[cbuf.at[j]], add=True)
        return carry

    lax.fori_loop(0, EPT // WIN, step, 0)
    plsc.subcore_barrier()
    pltpu.sync_copy(deg_sh.at[pl.ds(r0, RPT)], dout.at[c, pl.ds(r0, RPT)])


def _deg_call(col2d, ew2d, z16):
    return pl.kernel(
        _deg_body,
        out_type=jax.ShapeDtypeStruct((2, N, D), jnp.float32),
        mesh=_sc_mesh(),
        compiler_params=_sc_params,
        scratch_types=[
            pltpu.VMEM((SUB, CHUNK), jnp.int32),
            pltpu.VMEM((SUB, CHUNK), jnp.float32),
            pltpu.VMEM((CHUNK, D), jnp.float32),
            pltpu.VMEM_SHARED((N, D), jnp.float32),
        ],
    )(col2d, ew2d, z16)


def _agg_body(row2, col2d, ew2d, mptab, z16, out, rbuf, cbuf, wbuf,
              gb0, gb1, gb2, gb3, agg_sh, semi, semg, sems):
    c = lax.axis_index("c")
    s = lax.axis_index("s")
    r0 = s * RPT
    pltpu.sync_copy(z16.at[pl.ds(r0, RPT)], agg_sh.at[pl.ds(r0, RPT)])
    plsc.subcore_barrier()
    base = s * TPS
    gbs = [gb0, gb1, gb2, gb3]
    zero16 = lax.iota(jnp.int32, 16) * 0

    def step(st, carry):
        off = base + st * WIN
        q = off // CHUNK
        d0 = pltpu.async_copy(row2.at[c, pl.ds(q, SUB)], rbuf, semi)
        d1 = pltpu.async_copy(col2d.at[pl.ds(q, SUB)], cbuf, semi)
        d2 = pltpu.async_copy(ew2d.at[pl.ds(q, SUB)], wbuf, semi)
        d0.wait()
        d1.wait()
        d2.wait()
        gds = [pltpu.async_copy(mptab.at[rbuf.at[j]], gbs[j], semg)
               for j in range(SUB)]
        sds = []
        for j in range(SUB):
            gds[j].wait()
            gbuf = gbs[j]
            for e in range(CHUNK):
                ew16 = plsc.load_gather(wbuf, [zero16 + j, zero16 + e])
                gbuf[e, :] = gbuf[e, :] * ew16
            sds.append(pltpu.async_copy(gbuf, agg_sh.at[cbuf.at[j]], sems,
                                        add=True))
        for d in sds:
            d.wait()
        return carry

    lax.fori_loop(0, TPS // WIN, step, 0)
    plsc.subcore_barrier()
    pltpu.sync_copy(agg_sh.at[pl.ds(r0, RPT)], out.at[c, pl.ds(r0, RPT)])


def _agg_call(row2, col2d, ew2d, mptab, z16):
    return pl.kernel(
        _agg_body,
        out_type=jax.ShapeDtypeStruct((2, N, D), jnp.float32),
        mesh=_sc_mesh(),
        compiler_params=_sc_params,
        scratch_types=[
            pltpu.VMEM((SUB, CHUNK), jnp.int32),
            pltpu.VMEM((SUB, CHUNK), jnp.int32),
            pltpu.VMEM((SUB, CHUNK), jnp.float32),
            pltpu.VMEM((CHUNK, D), jnp.float32),
            pltpu.VMEM((CHUNK, D), jnp.float32),
            pltpu.VMEM((CHUNK, D), jnp.float32),
            pltpu.VMEM((CHUNK, D), jnp.float32),
            pltpu.VMEM_SHARED((N, D), jnp.float32),
            pltpu.SemaphoreType.DMA,
            pltpu.SemaphoreType.DMA,
            pltpu.SemaphoreType.DMA,
        ],
    )(row2, col2d, ew2d, mptab, z16)


_BE = 2000
_GE = N // _BE


def _dis_of(degp_blk):
    deg = degp_blk[0, :, 0:1] + degp_blk[1, :, 0:1]
    safe = jnp.where(deg > 0, deg, 1.0)
    return jnp.where(deg > 0, lax.rsqrt(safe), 0.0)


def _enc_body(xa, xb, wih, bih, whh, bhh, fcw, fcb, w1, w1r, b1, degp,
              mp_ref, hr_ref):
    x = jnp.concatenate([xa[...], xb[...]], axis=0)
    gi_all = jnp.dot(x, wih[0], preferred_element_type=jnp.float32) + bih[0]
    h = jnp.zeros((2 * _BE, H), jnp.float32)
    w = whh[0]
    bh = bhh[0]
    for t in range(L):
        gi = gi_all[:, 3 * H * t:3 * H * (t + 1)]
        gh = jnp.dot(h, w, preferred_element_type=jnp.float32) + bh
        r = jax.nn.sigmoid(gi[:, :H] + gh[:, :H])
        z = jax.nn.sigmoid(gi[:, H:2 * H] + gh[:, H:2 * H])
        n = jnp.tanh(gi[:, 2 * H:] + r * gh[:, 2 * H:])
        h = (1.0 - z) * n + z * h
    ha = h[:_BE]
    hb = h[_BE:]
    hfc = jax.nn.relu(jnp.dot(ha, fcw[0], preferred_element_type=jnp.float32)
                      + jnp.dot(hb, fcw[1], preferred_element_type=jnp.float32)
                      + fcb[...])
    dis = _dis_of(degp[...])
    mp = dis * jnp.dot(hfc, w1[...], preferred_element_type=jnp.float32)
    mp_ref[0] = mp[:, :D]
    mp_ref[1] = jnp.concatenate(
        [mp[:, D:], jnp.zeros((_BE, 2 * D - H), jnp.float32)], axis=1)
    hr_ref[...] = jnp.dot(hfc, w1r[...], preferred_element_type=jnp.float32) + b1[...]


def _enc_call(xa, xb, wih, bih, whh, bhh, fcw, fcb, w1, w1r, b1, degp):
    g2 = _GE // 2
    return pl.pallas_call(
        _enc_body,
        grid=(_GE,),
        in_specs=[
            pl.BlockSpec((_BE, 2 * L), lambda j: (j, 0)),
            pl.BlockSpec((_BE, 2 * L), lambda j: (j, 0)),
            pl.BlockSpec((1, 2 * L, 3 * H * L), lambda j: (j // g2, 0, 0)),
            pl.BlockSpec((1, 1, 3 * H * L), lambda j: (j // g2, 0, 0)),
            pl.BlockSpec((1, H, 3 * H), lambda j: (j // g2, 0, 0)),
            pl.BlockSpec((1, 1, 3 * H), lambda j: (j // g2, 0, 0)),
            pl.BlockSpec((2, H, H), lambda j: (0, 0, 0)),
            pl.BlockSpec((1, H), lambda j: (0, 0)),
            pl.BlockSpec((H, H), lambda j: (0, 0)),
            pl.BlockSpec((H, H), lambda j: (0, 0)),
            pl.BlockSpec((1, H), lambda j: (0, 0)),
            pl.BlockSpec((2, _BE, D), lambda j: (0, j, 0)),
        ],
        out_specs=[
            pl.BlockSpec((2, _BE, D), lambda j: (0, j, 0)),
            pl.BlockSpec((_BE, H), lambda j: (j, 0)),
        ],
        out_shape=[
            jax.ShapeDtypeStruct((2, N, D), jnp.float32),
            jax.ShapeDtypeStruct((N, H), jnp.float32),
        ],
    )(xa, xb, wih, bih, whh, bhh, fcw, fcb, w1, w1r, b1, degp)


def _agg20(aggp_blk):
    return jnp.concatenate([aggp_blk[0], aggp_blk[1][:, :H - D]], axis=1)


def _c1_body(aggp, degp, hr1, w2, w2r, b2, mp2_ref, hr2_ref):
    dis = _dis_of(degp[...])
    h1 = jax.nn.relu(dis * _agg20(aggp[...]) + hr1[...])
    mp2 = dis * jnp.dot(h1, w2[...], preferred_element_type=jnp.float32)
    mp2_ref[0] = mp2[:, :D]
    mp2_ref[1] = jnp.concatenate(
        [mp2[:, D:], jnp.zeros((_BE, 2 * D - H), jnp.float32)], axis=1)
    hr2_ref[...] = jnp.dot(h1, w2r[...], preferred_element_type=jnp.float32) + b2[...]


def _c1_call(aggp, degp, hr1, w2, w2r, b2):
    return pl.pallas_call(
        _c1_body,
        grid=(_GE,),
        in_specs=[
            pl.BlockSpec((2, _BE, D), lambda j: (0, j, 0)),
            pl.BlockSpec((2, _BE, D), lambda j: (0, j, 0)),
            pl.BlockSpec((_BE, H), lambda j: (j, 0)),
            pl.BlockSpec((H, H), lambda j: (0, 0)),
            pl.BlockSpec((H, H), lambda j: (0, 0)),
            pl.BlockSpec((1, H), lambda j: (0, 0)),
        ],
        out_specs=[
            pl.BlockSpec((2, _BE, D), lambda j: (0, j, 0)),
            pl.BlockSpec((_BE, H), lambda j: (j, 0)),
        ],
        out_shape=[
            jax.ShapeDtypeStruct((2, N, D), jnp.float32),
            jax.ShapeDtypeStruct((N, H), jnp.float32),
        ],
    )(aggp, degp, hr1, w2, w2r, b2)


def _c2_body(aggp, degp, hr2, out_ref):
    dis = _dis_of(degp[...])
    out_ref[...] = jax.nn.relu(dis * _agg20(aggp[...]) + hr2[...])


def _c2_call(aggp, degp, hr2):
    return pl.pallas_call(
        _c2_body,
        grid=(_GE,),
        in_specs=[
            pl.BlockSpec((2, _BE, D), lambda j: (0, j, 0)),
            pl.BlockSpec((2, _BE, D), lambda j: (0, j, 0)),
            pl.BlockSpec((_BE, H), lambda j: (j, 0)),
        ],
        out_specs=pl.BlockSpec((_BE, H), lambda j: (j, 0)),
        out_shape=jax.ShapeDtypeStruct((N, H), jnp.float32),
    )(aggp, degp, hr2)


def _build_gru_weights(w_ih_f, b_ih_f, w_ih_b, b_ih_b, w_hh_f, b_hh_f,
                       w_hh_b, b_hh_b):
    wih = jnp.zeros((2, 2 * L, 3 * H * L), jnp.float32)
    for t in range(L):
        wih = wih.at[0, 2 * t:2 * t + 2, 3 * H * t:3 * H * (t + 1)].set(w_ih_f.T)
        tb = L - 1 - t
        wih = wih.at[1, 2 * tb:2 * tb + 2, 3 * H * t:3 * H * (t + 1)].set(w_ih_b.T)
    bih = jnp.stack([jnp.tile(b_ih_f, L), jnp.tile(b_ih_b, L)])[:, None, :]
    whh = jnp.stack([w_hh_f.T, w_hh_b.T])
    bhh = jnp.stack([b_hh_f, b_hh_b])[:, None, :]
    return wih, bih, whh, bhh


def kernel(x, edge_index, edge_attr, w_ih_f, w_hh_f, b_ih_f, b_hh_f,
           w_ih_b, w_hh_b, b_ih_b, b_hh_b, fc_w, fc_b,
           w1_init, w1_root, b1, w2_init, w2_root, b2, data):
    edge_index = edge_index.astype(jnp.int32)
    rowp = jnp.concatenate([edge_index[0], jnp.zeros((EPAD - E,), jnp.int32)])
    colp = jnp.concatenate([edge_index[1], jnp.zeros((EPAD - E,), jnp.int32)])
    ewp = jnp.concatenate([edge_attr, jnp.zeros((EPAD - E,), jnp.float32)])
    row2 = jnp.stack([rowp, rowp + N]).reshape(
        2, EPAD // CHUNK, CHUNK)
    col2d = colp.reshape(EPAD // CHUNK, CHUNK)
    ew2d = ewp.reshape(EPAD // CHUNK, CHUNK)

    x2d = x.reshape(N, 2 * L)
    xe = x2d[0::2]
    xo = x2d[1::2]
    xa = jnp.concatenate([xe, xe], axis=0)
    xb = jnp.concatenate([xo, xo], axis=0)

    wih, bih, whh, bhh = _build_gru_weights(
        w_ih_f, b_ih_f, w_ih_b, b_ih_b, w_hh_f, b_hh_f, w_hh_b, b_hh_b)
    fcw = jnp.stack([fc_w.T[:H], fc_w.T[H:]])
    fcb = fc_b[None, :]

    z16 = jnp.zeros((N, D), jnp.float32)

    degp = _deg_call(col2d, ew2d, z16)
    mp1, hr1 = _enc_call(xa, xb, wih, bih, whh, bhh, fcw, fcb,
                         w1_init, w1_root, b1[None, :], degp)
    aggp1 = _agg_call(row2, col2d, ew2d, mp1.reshape(2 * N, D), z16)
    mp2, hr2 = _c1_call(aggp1, degp, hr1, w2_init, w2_root, b2[None, :])
    aggp2 = _agg_call(row2, col2d, ew2d, mp2.reshape(2 * N, D), z16)
    return _c2_call(aggp2, degp, hr2)
